# baseline (speedup 1.0000x reference)
import jax
import jax.numpy as jnp
from jax.experimental import pallas as pl
from jax.experimental.pallas import tpu as pltpu

_NUM_RANKS = 1024
_MAX_TOK = 77
_D = 768
_CTX = 16
_TPR = 4
_RB = 32


def _body(ctx_ref, rank_ref, sent_ref, out_ref):
    out_ref[:, 0:1, :] = sent_ref[:, 0:1, :]
    out_ref[:, 1:1 + _CTX, :] = jnp.broadcast_to(ctx_ref[...][None],
                                                 (_RB, _CTX, _D))
    out_ref[:, 1 + _CTX:1 + _CTX + _TPR, :] = rank_ref[...]
    tail = 1 + _CTX + _TPR
    out_ref[:, tail:, :] = sent_ref[:, tail:, :]


def kernel(context_embeds, rank_embeds, sentence_embeds):
    return pl.pallas_call(
        _body,
        grid=(_NUM_RANKS // _RB,),
        in_specs=[
            pl.BlockSpec((_CTX, _D), lambda i: (0, 0)),
            pl.BlockSpec((_RB, _TPR, _D), lambda i: (i, 0, 0)),
            pl.BlockSpec((_RB, _MAX_TOK, _D), lambda i: (i, 0, 0)),
        ],
        out_specs=pl.BlockSpec((_RB, _MAX_TOK, _D), lambda i: (i, 0, 0)),
        out_shape=jax.ShapeDtypeStruct((_NUM_RANKS, _MAX_TOK, _D),
                                       jnp.float32),
        compiler_params=pltpu.CompilerParams(
            dimension_semantics=("parallel",)),
    )(context_embeds, rank_embeds, sentence_embeds)


# speedup vs baseline: 1.0050x; 1.0050x over previous
import jax
import jax.numpy as jnp
from jax.experimental import pallas as pl
from jax.experimental.pallas import tpu as pltpu

_NUM_RANKS = 1024
_MAX_TOK = 77
_D = 768
_CTX = 16
_TPR = 4

_CR = 16
_NBUF = 4
_L = 2
_NCHUNK = _NUM_RANKS // _CR


def _in_copy(sent_hbm, buf, in_sem, chunk, slot):
    return pltpu.make_async_copy(
        sent_hbm.at[pl.ds(chunk * _CR, _CR)], buf.at[slot], in_sem.at[slot])


def _rk_copy(rank_hbm, rbuf, rk_sem, chunk, slot):
    return pltpu.make_async_copy(
        rank_hbm.at[pl.ds(chunk * _CR, _CR)], rbuf.at[slot], rk_sem.at[slot])


def _out_copy(buf, out_hbm, out_sem, chunk, slot):
    return pltpu.make_async_copy(
        buf.at[slot], out_hbm.at[pl.ds(chunk * _CR, _CR)], out_sem.at[slot])


def _body(ctx_ref, rank_hbm, sent_hbm, out_hbm, buf, rbuf, in_sem, rk_sem,
          out_sem):
    i = pl.program_id(0)
    slot = jax.lax.rem(i, _NBUF)

    @pl.when(i == 0)
    def _prime():
        for j in range(_L):
            _in_copy(sent_hbm, buf, in_sem, j, j).start()
            _rk_copy(rank_hbm, rbuf, rk_sem, j, j).start()

    @pl.when(i + _L < _NCHUNK)
    def _lookahead():
        nxt = i + _L
        slot2 = jax.lax.rem(nxt, _NBUF)

        @pl.when(nxt >= _NBUF)
        def _reclaim():
            _out_copy(buf, out_hbm, out_sem, nxt - _NBUF, slot2).wait()

        _in_copy(sent_hbm, buf, in_sem, nxt, slot2).start()
        _rk_copy(rank_hbm, rbuf, rk_sem, nxt, slot2).start()

    _in_copy(sent_hbm, buf, in_sem, i, slot).wait()
    _rk_copy(rank_hbm, rbuf, rk_sem, i, slot).wait()

    b = buf.at[slot]
    b[:, 1:1 + _CTX, :] = jnp.broadcast_to(ctx_ref[...][None],
                                           (_CR, _CTX, _D))
    b[:, 1 + _CTX:1 + _CTX + _TPR, :] = rbuf[slot]

    _out_copy(buf, out_hbm, out_sem, i, slot).start()

    @pl.when(i == _NCHUNK - 1)
    def _drain():
        for d in range(min(_NBUF, _NCHUNK)):
            chunk = _NCHUNK - 1 - d
            _out_copy(buf, out_hbm, out_sem, chunk, chunk % _NBUF).wait()


def kernel(context_embeds, rank_embeds, sentence_embeds):
    return pl.pallas_call(
        _body,
        grid=(_NCHUNK,),
        in_specs=[
            pl.BlockSpec((_CTX, _D), lambda i: (0, 0)),
            pl.BlockSpec(memory_space=pltpu.MemorySpace.HBM),
            pl.BlockSpec(memory_space=pltpu.MemorySpace.HBM),
        ],
        out_specs=pl.BlockSpec(memory_space=pltpu.MemorySpace.HBM),
        out_shape=jax.ShapeDtypeStruct((_NUM_RANKS, _MAX_TOK, _D),
                                       jnp.float32),
        scratch_shapes=[
            pltpu.VMEM((_NBUF, _CR, _MAX_TOK, _D), jnp.float32),
            pltpu.VMEM((_NBUF, _CR, _TPR, _D), jnp.float32),
            pltpu.SemaphoreType.DMA((_NBUF,)),
            pltpu.SemaphoreType.DMA((_NBUF,)),
            pltpu.SemaphoreType.DMA((_NBUF,)),
        ],
        compiler_params=pltpu.CompilerParams(
            dimension_semantics=("arbitrary",)),
    )(context_embeds, rank_embeds, sentence_embeds)
